# Initial kernel scaffold; baseline (speedup 1.0000x reference)
import functools

import jax
import jax.numpy as jnp
from jax import lax
from jax.experimental import pallas as pl
from jax.experimental.pallas import tpu as pltpu
from jax.experimental.pallas import tpu_sc as plsc

NC, NS, L = 2, 16, 16
NW = NC * NS
HID = 128
HEADS = 4
F = HEADS * HID


def _gine_edges(x, src, dst, ea):
    N = x.shape[0]
    E = src.shape[0]
    EPT = E // NW
    K = 80
    CH = EPT // K
    ROWS = N // NS
    zeros = jnp.zeros((N, HID), jnp.float32)
    mesh = plsc.VectorSubcoreMesh(core_axis_name="c", subcore_axis_name="s",
                                  num_cores=NC, num_subcores=NS)

    @functools.partial(
        pl.kernel, mesh=mesh,
        out_type=jax.ShapeDtypeStruct((NC, N, HID), jnp.float32),
        scratch_types=[
            pltpu.VMEM((K,), jnp.int32),
            pltpu.VMEM((K,), jnp.int32),
            pltpu.VMEM((K, HID), jnp.float32),
            pltpu.VMEM((K, HID), jnp.float32),
            pltpu.VMEM_SHARED((N, HID), jnp.float32),
            pltpu.SemaphoreType.DMA,
        ])
    def kern(x_h, src_h, dst_h, ea_h, z_h, out_h, src_v, dst_v, xg, eav, acc, sem):
        c = lax.axis_index("c")
        s = lax.axis_index("s")
        tid = c * NS + s
        pltpu.sync_copy(z_h.at[pl.ds(s * ROWS, ROWS)], acc.at[pl.ds(s * ROWS, ROWS)])
        plsc.subcore_barrier()
        base0 = tid * EPT

        def chunk(i, _):
            b = base0 + i * K
            pltpu.sync_copy(src_h.at[pl.ds(b, K)], src_v)
            pltpu.sync_copy(dst_h.at[pl.ds(b, K)], dst_v)
            pltpu.async_copy(x_h.at[src_v], xg, sem).wait()
            pltpu.sync_copy(ea_h.at[pl.ds(b, K)], eav)

            def row(r, _):
                for cc in range(HID // L):
                    sl = pl.ds(cc * L, L)
                    xg[r, sl] = jnp.maximum(xg[r, sl] + eav[r, sl], 0.0)
                return 0

            lax.fori_loop(0, K, row, 0)
            pltpu.sync_copy(xg, acc.at[dst_v], add=True)
            return 0

        lax.fori_loop(0, CH, chunk, 0)
        plsc.subcore_barrier()
        pltpu.sync_copy(acc.at[pl.ds(s * ROWS, ROWS)],
                        out_h.at[c, pl.ds(s * ROWS, ROWS)])

    return kern(x, src, dst, ea, zeros)


def _conv_mlp(x, parts, mp):
    N = x.shape[0]

    def body(x_r, p_r, e_r, W1, b1, g1, be1, W2, b2, g2, be2, o_r):
        z = (1.0 + e_r[0, 0]) * x_r[...] + p_r[0] + p_r[1]
        h = jnp.dot(z, W1[...], preferred_element_type=jnp.float32) + b1[...]
        m = jnp.mean(h, axis=0, keepdims=True)
        v = jnp.mean((h - m) ** 2, axis=0, keepdims=True)
        h = jnp.maximum(g1[...] * (h - m) / jnp.sqrt(v + 1e-5) + be1[...], 0.0)
        h = jnp.dot(h, W2[...], preferred_element_type=jnp.float32) + b2[...]
        m = jnp.mean(h, axis=0, keepdims=True)
        v = jnp.mean((h - m) ** 2, axis=0, keepdims=True)
        h = jnp.maximum(g2[...] * (h - m) / jnp.sqrt(v + 1e-5) + be2[...], 0.0)
        o_r[...] = h + x_r[...]

    p = mp['mlp']
    return pl.pallas_call(
        body, out_shape=jax.ShapeDtypeStruct((N, HID), jnp.float32),
    )(x, parts, jnp.reshape(1.0 * mp['eps'], (1, 1)),
      p['W1'], p['b1'][None], p['g1'][None], p['be1'][None],
      p['W2'], p['b2'][None], p['g2'][None], p['be2'][None])


def _gat_prep(xs, xd, Ws, Wd, a_s, a_d):
    ns, nd = xs.shape[0], xd.shape[0]

    def body(xs_r, xd_r, Ws_r, Wd_r, as_r, ad_r, hs0, hs1, hs2, hs3,
             asr_o, adr_o, g_o):
        xsv = xs_r[...]
        xdv = xd_r[...]
        Wsv = Ws_r[...]
        Wdv = Wd_r[...]
        zrow = jnp.zeros((8, HID), jnp.float32)
        for h, hs_o in enumerate((hs0, hs1, hs2, hs3)):
            blk = jnp.dot(xsv, Wsv[:, h * HID:(h + 1) * HID],
                          preferred_element_type=jnp.float32)
            hs_o[...] = jnp.concatenate([blk, zrow], axis=0)
        wa = [jnp.dot(Wsv[:, h * HID:(h + 1) * HID], as_r[h]) for h in range(HEADS)]
        wd = [jnp.dot(Wdv[:, h * HID:(h + 1) * HID], ad_r[h]) for h in range(HEADS)]
        Wa = jnp.concatenate([jnp.stack(wa, axis=1),
                              jnp.zeros((HID, 16 - HEADS), jnp.float32)], axis=1)
        Wd16 = jnp.concatenate([jnp.stack(wd, axis=1),
                                jnp.zeros((HID, 16 - HEADS), jnp.float32)], axis=1)
        asrc = jnp.dot(xsv, Wa, preferred_element_type=jnp.float32)
        adst = jnp.dot(xdv, Wd16, preferred_element_type=jnp.float32)
        asr_o[...] = jnp.concatenate(
            [asrc, jnp.full((8, 16), -1e30, jnp.float32)], axis=0)
        adr_o[...] = adst
        y = jnp.max(asrc, axis=0, keepdims=True) + jnp.max(adst, axis=0,
                                                           keepdims=True)
        g_o[...] = jnp.where(y > 0, y, 0.2 * y)

    return pl.pallas_call(
        body,
        out_shape=(
            jax.ShapeDtypeStruct((ns + 8, HID), jnp.float32),
            jax.ShapeDtypeStruct((ns + 8, HID), jnp.float32),
            jax.ShapeDtypeStruct((ns + 8, HID), jnp.float32),
            jax.ShapeDtypeStruct((ns + 8, HID), jnp.float32),
            jax.ShapeDtypeStruct((ns + 8, 16), jnp.float32),
            jax.ShapeDtypeStruct((nd, 16), jnp.float32),
            jax.ShapeDtypeStruct((1, 16), jnp.float32),
        ),
    )(xs, xd, Ws, Wd, a_s, a_d)


def _gat_att(srcp, dstp, asrc16, adst16, g16, nd):
    Ep = srcp.shape[0]
    EPT = Ep // NW
    K = 64
    CH = EPT // K
    RZ = nd // NS
    zeros = jnp.zeros((nd, 16), jnp.float32)
    mesh = plsc.VectorSubcoreMesh(core_axis_name="c", subcore_axis_name="s",
                                  num_cores=NC, num_subcores=NS)

    @functools.partial(
        pl.kernel, mesh=mesh,
        out_type=(jax.ShapeDtypeStruct((Ep, 16), jnp.float32),
                  jax.ShapeDtypeStruct((NC, nd, 16), jnp.float32)),
        scratch_types=[
            pltpu.VMEM((K,), jnp.int32),
            pltpu.VMEM((K,), jnp.int32),
            pltpu.VMEM((K, 16), jnp.float32),
            pltpu.VMEM((K, 16), jnp.float32),
            pltpu.VMEM((K, 16), jnp.float32),
            pltpu.VMEM((16,), jnp.float32),
            pltpu.VMEM_SHARED((nd, 16), jnp.float32),
            pltpu.SemaphoreType.DMA,
        ])
    def kern(src_h, dst_h, asrc_h, adst_h, g_h, z_h, ex_h, den_h,
             src_v, dst_v, sg, dg, exb, gv, den, sem):
        c = lax.axis_index("c")
        s = lax.axis_index("s")
        tid = c * NS + s
        pltpu.sync_copy(g_h, gv)
        pltpu.sync_copy(z_h.at[pl.ds(s * RZ, RZ)], den.at[pl.ds(s * RZ, RZ)])
        plsc.subcore_barrier()
        gvec = gv[...]
        base0 = tid * EPT

        def chunk(i, _):
            b = base0 + i * K
            pltpu.sync_copy(src_h.at[pl.ds(b, K)], src_v)
            pltpu.sync_copy(dst_h.at[pl.ds(b, K)], dst_v)
            pltpu.async_copy(asrc_h.at[src_v], sg, sem).wait()
            pltpu.async_copy(adst_h.at[dst_v], dg, sem).wait()

            def row(r, _):
                y = sg[r, :] + dg[r, :]
                al = jnp.where(y > 0, y, 0.2 * y)
                exb[r, :] = jnp.exp(al - gvec)
                return 0

            lax.fori_loop(0, K, row, 0)
            pltpu.sync_copy(exb, ex_h.at[pl.ds(b, K)])
            pltpu.sync_copy(exb, den.at[dst_v], add=True)
            return 0

        lax.fori_loop(0, CH, chunk, 0)
        plsc.subcore_barrier()
        pltpu.sync_copy(den.at[pl.ds(s * RZ, RZ)], den_h.at[c, pl.ds(s * RZ, RZ)])

    return kern(srcp, dstp, asrc16, adst16, g16, zeros)


def _gat_scatter(srcp, dstp, ex, den0, den1, hs_list, nd):
    Ep = srcp.shape[0]
    EPT = Ep // NW
    K = 64
    CH = EPT // K
    RZ = nd // NS
    zeros = jnp.zeros((nd, HID), jnp.float32)
    mesh = plsc.VectorSubcoreMesh(core_axis_name="c", subcore_axis_name="s",
                                  num_cores=NC, num_subcores=NS)

    @functools.partial(
        pl.kernel, mesh=mesh,
        out_type=jax.ShapeDtypeStruct((NC, HEADS, nd, HID), jnp.float32),
        scratch_types=[
            pltpu.VMEM((K,), jnp.int32),
            pltpu.VMEM((K,), jnp.int32),
            pltpu.VMEM((K, 16), jnp.float32),
            pltpu.VMEM((K, 16), jnp.float32),
            pltpu.VMEM((K, 16), jnp.float32),
            pltpu.VMEM((K, HID), jnp.float32),
            pltpu.VMEM((K, HID), jnp.float32),
            pltpu.VMEM_SHARED((nd, HID), jnp.float32),
            pltpu.SemaphoreType.DMA,
        ])
    def kern(src_h, dst_h, ex_h, d0_h, d1_h, hs0, hs1, hs2, hs3, z_h, out_h,
             src_v, dst_v, exv, d0, d1, hg, ob, acc, sem):
        c = lax.axis_index("c")
        s = lax.axis_index("s")
        tid = c * NS + s
        base0 = tid * EPT
        lanes = lax.iota(jnp.int32, 16)
        hs_h = (hs0, hs1, hs2, hs3)
        for h in range(HEADS):
            pltpu.sync_copy(z_h.at[pl.ds(s * RZ, RZ)], acc.at[pl.ds(s * RZ, RZ)])
            plsc.subcore_barrier()

            def chunk(i, _, h=h):
                b = base0 + i * K
                pltpu.sync_copy(src_h.at[pl.ds(b, K)], src_v)
                pltpu.sync_copy(dst_h.at[pl.ds(b, K)], dst_v)
                pltpu.sync_copy(ex_h.at[pl.ds(b, K)], exv)
                pltpu.async_copy(d0_h.at[dst_v], d0, sem).wait()
                pltpu.async_copy(d1_h.at[dst_v], d1, sem).wait()
                pltpu.async_copy(hs_h[h].at[src_v], hg, sem).wait()

                def row(r, _, h=h):
                    den = jnp.maximum(d0[r, :] + d1[r, :], 1e-30)
                    w = exv[r, :] / den
                    ws = jnp.sum(jnp.where(lanes == h, w, 0.0))
                    for cc in range(HID // L):
                        sl = pl.ds(cc * L, L)
                        ob[r, sl] = hg[r, sl] * ws
                    return 0

                lax.fori_loop(0, K, row, 0)
                pltpu.sync_copy(ob, acc.at[dst_v], add=True)
                return 0

            lax.fori_loop(0, CH, chunk, 0)
            plsc.subcore_barrier()
            pltpu.sync_copy(acc.at[pl.ds(s * RZ, RZ)],
                            out_h.at[c, h, pl.ds(s * RZ, RZ)])
            plsc.subcore_barrier()

    return kern(srcp, dstp, ex, den0, den1, *hs_list, zeros)


def _gat_mlp(parts, gb, mp, res):
    nd = res.shape[0]

    def body(p_r, gb_r, W1, b1, g1, be1, W2, b2, g2, be2, res_r, o_r):
        o = jnp.concatenate([p_r[0, h] + p_r[1, h] for h in range(HEADS)],
                            axis=-1) + gb_r[...]
        h = jnp.dot(o, W1[...], preferred_element_type=jnp.float32) + b1[...]
        m = jnp.mean(h, axis=0, keepdims=True)
        v = jnp.mean((h - m) ** 2, axis=0, keepdims=True)
        h = jnp.maximum(g1[...] * (h - m) / jnp.sqrt(v + 1e-5) + be1[...], 0.0)
        h = jnp.dot(h, W2[...], preferred_element_type=jnp.float32) + b2[...]
        m = jnp.mean(h, axis=0, keepdims=True)
        v = jnp.mean((h - m) ** 2, axis=0, keepdims=True)
        h = jnp.maximum(g2[...] * (h - m) / jnp.sqrt(v + 1e-5) + be2[...], 0.0)
        o_r[...] = h + res_r[...]

    return pl.pallas_call(
        body, out_shape=jax.ShapeDtypeStruct((nd, HID), jnp.float32),
    )(parts, gb[None],
      mp['W1'], mp['b1'][None], mp['g1'][None], mp['be1'][None],
      mp['W2'], mp['b2'][None], mp['g2'][None], mp['be2'][None], res)


def _gat_block(xs, xd, ei, gp, mlp_p, res):
    ns = xs.shape[0]
    E = ei.shape[1]
    nd = xd.shape[0]
    Ep = ((E + NW * 64 - 1) // (NW * 64)) * (NW * 64)
    if Ep == E:
        Ep += NW * 64
    pad = Ep - E
    srcp = jnp.concatenate([ei[0], jnp.full((pad,), ns, jnp.int32)])
    dstp = jnp.concatenate([ei[1], jnp.zeros((pad,), jnp.int32)])
    hs0, hs1, hs2, hs3, asrc16, adst16, g16 = _gat_prep(
        xs, xd, gp['Ws'], gp['Wd'], gp['as'], gp['ad'])
    ex, den = _gat_att(srcp, dstp, asrc16, adst16, jnp.reshape(g16, (16,)), nd)
    parts = _gat_scatter(srcp, dstp, ex, den[0], den[1],
                         (hs0, hs1, hs2, hs3), nd)
    return _gat_mlp(parts, gp['b'], mlp_p, res)


def kernel(x, edge_index, edge_attr, x_cl, c2c_edge_index, c2c_edge_attr,
           atom2c_edge_index, c2atom_edge_index, params):
    src = edge_index[0]
    dst = edge_index[1]
    for p in params['atom_convs']:
        parts = _gine_edges(x, src, dst, edge_attr)
        x = _conv_mlp(x, parts, p)
    h = _gat_block(x_cl, x, c2atom_edge_index, params['unpool'],
                   params['c2atom_mlp'], x)
    h_cl = _gat_block(x, x_cl, atom2c_edge_index, params['pool'],
                      params['atom2c_mlp'], x_cl)
    return (h, h_cl)


# speedup vs baseline: 2.9089x; 2.9089x over previous
import functools

import jax
import jax.numpy as jnp
from jax import lax
from jax.experimental import pallas as pl
from jax.experimental.pallas import tpu as pltpu
from jax.experimental.pallas import tpu_sc as plsc

NC, NS, L = 2, 16, 16
NW = NC * NS
HID = 128
HEADS = 4
F = HEADS * HID


def _gine_edges(x, src, dst, ea):
    N = x.shape[0]
    E = src.shape[0]
    EPT = E // NW
    K = 80
    CH = EPT // K
    NP = -(-N // (NS * 8)) * (NS * 8)
    ROWS = NP // NS
    zeros = jnp.zeros((NP, HID), jnp.float32)
    mesh = plsc.VectorSubcoreMesh(core_axis_name="c", subcore_axis_name="s",
                                  num_cores=NC, num_subcores=NS)

    @functools.partial(
        pl.kernel, mesh=mesh,
        out_type=jax.ShapeDtypeStruct((NC, NP, HID), jnp.float32),
        scratch_types=[
            pltpu.VMEM((K,), jnp.int32),
            pltpu.VMEM((K,), jnp.int32),
            pltpu.VMEM((K, HID), jnp.float32),
            pltpu.VMEM((K, HID), jnp.float32),
            pltpu.VMEM_SHARED((NP, HID), jnp.float32),
            pltpu.SemaphoreType.DMA,
        ])
    def kern(x_h, src_h, dst_h, ea_h, z_h, out_h, src_v, dst_v, xg, eav, acc, sem):
        c = lax.axis_index("c")
        s = lax.axis_index("s")
        tid = c * NS + s
        pltpu.sync_copy(z_h.at[pl.ds(s * ROWS, ROWS)], acc.at[pl.ds(s * ROWS, ROWS)])
        plsc.subcore_barrier()
        base0 = tid * EPT

        def chunk(i, _):
            b = base0 + i * K
            pltpu.sync_copy(src_h.at[pl.ds(b, K)], src_v)
            pltpu.sync_copy(dst_h.at[pl.ds(b, K)], dst_v)
            pltpu.async_copy(x_h.at[src_v], xg, sem).wait()
            pltpu.sync_copy(ea_h.at[pl.ds(b, K)], eav)

            def row(r, _):
                for cc in range(HID // L):
                    sl = pl.ds(cc * L, L)
                    xg[r, sl] = jnp.maximum(xg[r, sl] + eav[r, sl], 0.0)
                return 0

            lax.fori_loop(0, K, row, 0)
            pltpu.sync_copy(xg, acc.at[dst_v], add=True)
            return 0

        lax.fori_loop(0, CH, chunk, 0)
        plsc.subcore_barrier()
        pltpu.sync_copy(acc.at[pl.ds(s * ROWS, ROWS)],
                        out_h.at[c, pl.ds(s * ROWS, ROWS)])

    return kern(x, src, dst, ea, zeros)


def _conv_mlp(x, parts, mp):
    N = x.shape[0]

    def body(x_r, p_r, e_r, W1, b1, g1, be1, W2, b2, g2, be2, o_r):
        z = (1.0 + e_r[0, 0]) * x_r[...] + p_r[0, :N] + p_r[1, :N]
        h = jnp.dot(z, W1[...], preferred_element_type=jnp.float32) + b1[...]
        m = jnp.mean(h, axis=0, keepdims=True)
        v = jnp.mean((h - m) ** 2, axis=0, keepdims=True)
        h = jnp.maximum(g1[...] * (h - m) / jnp.sqrt(v + 1e-5) + be1[...], 0.0)
        h = jnp.dot(h, W2[...], preferred_element_type=jnp.float32) + b2[...]
        m = jnp.mean(h, axis=0, keepdims=True)
        v = jnp.mean((h - m) ** 2, axis=0, keepdims=True)
        h = jnp.maximum(g2[...] * (h - m) / jnp.sqrt(v + 1e-5) + be2[...], 0.0)
        o_r[...] = h + x_r[...]

    p = mp['mlp']
    return pl.pallas_call(
        body, out_shape=jax.ShapeDtypeStruct((N, HID), jnp.float32),
    )(x, parts, jnp.reshape(mp['eps'], (1, 1)),
      p['W1'], p['b1'][None], p['g1'][None], p['be1'][None],
      p['W2'], p['b2'][None], p['g2'][None], p['be2'][None])


def _gat_prep(xs, xd, Ws, Wd, a_s, a_d):
    ns, nd = xs.shape[0], xd.shape[0]

    def body(xs_r, xd_r, Ws_r, Wd_r, as_r, ad_r, hs0, hs1, hs2, hs3,
             asr_o, adr_o, g_o):
        xsv = xs_r[...]
        xdv = xd_r[...]
        Wsv = Ws_r[...]
        Wdv = Wd_r[...]
        zrow = jnp.zeros((8, HID), jnp.float32)
        for h, hs_o in enumerate((hs0, hs1, hs2, hs3)):
            blk = jnp.dot(xsv, Wsv[:, h * HID:(h + 1) * HID],
                          preferred_element_type=jnp.float32)
            hs_o[...] = jnp.concatenate([blk, zrow], axis=0)
        zc = jnp.zeros((HID, 16 - HEADS), jnp.float32)
        wa = [jnp.dot(Wsv[:, h * HID:(h + 1) * HID],
                      jnp.reshape(as_r[h, :], (HID, 1)),
                      preferred_element_type=jnp.float32) for h in range(HEADS)]
        wd = [jnp.dot(Wdv[:, h * HID:(h + 1) * HID],
                      jnp.reshape(ad_r[h, :], (HID, 1)),
                      preferred_element_type=jnp.float32) for h in range(HEADS)]
        Wa = jnp.concatenate(wa + [zc], axis=1)
        Wd16 = jnp.concatenate(wd + [zc], axis=1)
        asrc = jnp.dot(xsv, Wa, preferred_element_type=jnp.float32)
        adst = jnp.dot(xdv, Wd16, preferred_element_type=jnp.float32)
        asr_o[...] = jnp.concatenate(
            [jnp.concatenate([asrc, jnp.zeros((ns, HID - 16), jnp.float32)],
                             axis=1),
             jnp.full((8, HID), -1e30, jnp.float32)], axis=0)
        adr_o[...] = jnp.concatenate(
            [adst, jnp.zeros((nd, HID - 16), jnp.float32)], axis=1)
        y = jnp.max(asrc, axis=0, keepdims=True) + jnp.max(adst, axis=0,
                                                           keepdims=True)
        g_o[...] = jnp.where(y > 0, y, 0.2 * y)

    return pl.pallas_call(
        body,
        out_shape=(
            jax.ShapeDtypeStruct((ns + 8, HID), jnp.float32),
            jax.ShapeDtypeStruct((ns + 8, HID), jnp.float32),
            jax.ShapeDtypeStruct((ns + 8, HID), jnp.float32),
            jax.ShapeDtypeStruct((ns + 8, HID), jnp.float32),
            jax.ShapeDtypeStruct((ns + 8, HID), jnp.float32),
            jax.ShapeDtypeStruct((nd, HID), jnp.float32),
            jax.ShapeDtypeStruct((1, 16), jnp.float32),
        ),
    )(xs, xd, Ws, Wd, a_s, a_d)


def _gat_att(srcp, dstp, asrc16, adst16, g16, nd):
    Ep = srcp.shape[0]
    EPT = Ep // NW
    K = 64
    CH = EPT // K
    ndp = -(-nd // (NS * 8)) * (NS * 8)
    RZ = ndp // NS
    zeros = jnp.zeros((ndp, HID), jnp.float32)
    mesh = plsc.VectorSubcoreMesh(core_axis_name="c", subcore_axis_name="s",
                                  num_cores=NC, num_subcores=NS)

    @functools.partial(
        pl.kernel, mesh=mesh,
        out_type=(jax.ShapeDtypeStruct((Ep, HID), jnp.float32),
                  jax.ShapeDtypeStruct((NC, ndp, HID), jnp.float32)),
        scratch_types=[
            pltpu.VMEM((K,), jnp.int32),
            pltpu.VMEM((K,), jnp.int32),
            pltpu.VMEM((K, HID), jnp.float32),
            pltpu.VMEM((K, HID), jnp.float32),
            pltpu.VMEM((K, HID), jnp.float32),
            pltpu.VMEM((16,), jnp.float32),
            pltpu.VMEM_SHARED((ndp, HID), jnp.float32),
            pltpu.SemaphoreType.DMA,
        ])
    def kern(src_h, dst_h, asrc_h, adst_h, g_h, z_h, ex_h, den_h,
             src_v, dst_v, sg, dg, exb, gv, den, sem):
        c = lax.axis_index("c")
        s = lax.axis_index("s")
        tid = c * NS + s
        pltpu.sync_copy(g_h, gv)
        pltpu.sync_copy(z_h.at[pl.ds(s * RZ, RZ)], den.at[pl.ds(s * RZ, RZ)])
        pltpu.sync_copy(z_h.at[pl.ds(0, K)], exb)
        plsc.subcore_barrier()
        gvec = gv[...]
        base0 = tid * EPT

        def chunk(i, _):
            b = base0 + i * K
            pltpu.sync_copy(src_h.at[pl.ds(b, K)], src_v)
            pltpu.sync_copy(dst_h.at[pl.ds(b, K)], dst_v)
            pltpu.async_copy(asrc_h.at[src_v], sg, sem).wait()
            pltpu.async_copy(adst_h.at[dst_v], dg, sem).wait()

            def row(r, _):
                y = sg[r, pl.ds(0, 16)] + dg[r, pl.ds(0, 16)]
                al = jnp.where(y > 0, y, 0.2 * y)
                exb[r, pl.ds(0, 16)] = jnp.exp(al - gvec)
                return 0

            lax.fori_loop(0, K, row, 0)
            pltpu.sync_copy(exb, ex_h.at[pl.ds(b, K)])
            pltpu.sync_copy(exb, den.at[dst_v], add=True)
            return 0

        lax.fori_loop(0, CH, chunk, 0)
        plsc.subcore_barrier()
        pltpu.sync_copy(den.at[pl.ds(s * RZ, RZ)], den_h.at[c, pl.ds(s * RZ, RZ)])

    return kern(srcp, dstp, asrc16, adst16, g16, zeros)


def _gat_scatter(srcp, dstp, ex, den0, den1, hs_list, nd):
    Ep = srcp.shape[0]
    EPT = Ep // NW
    K = 64
    CH = EPT // K
    ndp = -(-nd // (NS * 8)) * (NS * 8)
    RZ = ndp // NS
    zeros = jnp.zeros((ndp, HID), jnp.float32)
    mesh = plsc.VectorSubcoreMesh(core_axis_name="c", subcore_axis_name="s",
                                  num_cores=NC, num_subcores=NS)

    @functools.partial(
        pl.kernel, mesh=mesh,
        out_type=jax.ShapeDtypeStruct((NC, HEADS, ndp, HID), jnp.float32),
        scratch_types=[
            pltpu.VMEM((K,), jnp.int32),
            pltpu.VMEM((K,), jnp.int32),
            pltpu.VMEM((K, HID), jnp.float32),
            pltpu.VMEM((K, HID), jnp.float32),
            pltpu.VMEM((K, HID), jnp.float32),
            pltpu.VMEM((K, HID), jnp.float32),
            pltpu.VMEM((K, HID), jnp.float32),
            pltpu.VMEM((K, 16), jnp.float32),
            pltpu.VMEM_SHARED((ndp, HID), jnp.float32),
            pltpu.SemaphoreType.DMA,
        ])
    def kern(src_h, dst_h, ex_h, d0_h, d1_h, hs0, hs1, hs2, hs3, z_h, out_h,
             src_v, dst_v, exv, d0, d1, hg, ob, wb, acc, sem):
        c = lax.axis_index("c")
        s = lax.axis_index("s")
        tid = c * NS + s
        base0 = tid * EPT
        hs_h = (hs0, hs1, hs2, hs3)
        for h in range(HEADS):
            pltpu.sync_copy(z_h.at[pl.ds(s * RZ, RZ)], acc.at[pl.ds(s * RZ, RZ)])
            plsc.subcore_barrier()

            def chunk(i, _, h=h):
                b = base0 + i * K
                pltpu.sync_copy(src_h.at[pl.ds(b, K)], src_v)
                pltpu.sync_copy(dst_h.at[pl.ds(b, K)], dst_v)
                pltpu.sync_copy(ex_h.at[pl.ds(b, K)], exv)
                pltpu.async_copy(d0_h.at[dst_v], d0, sem).wait()
                pltpu.async_copy(d1_h.at[dst_v], d1, sem).wait()
                pltpu.async_copy(hs_h[h].at[src_v], hg, sem).wait()

                def row(r, _, h=h):
                    den = jnp.maximum(d0[r, pl.ds(0, 16)] + d1[r, pl.ds(0, 16)],
                                      1e-30)
                    w = exv[r, pl.ds(0, 16)] / den
                    ws = w[h]
                    for cc in range(HID // L):
                        sl = pl.ds(cc * L, L)
                        ob[r, sl] = hg[r, sl] * ws
                    return 0

                lax.fori_loop(0, K, row, 0)
                pltpu.sync_copy(ob, acc.at[dst_v], add=True)
                return 0

            lax.fori_loop(0, CH, chunk, 0)
            plsc.subcore_barrier()
            pltpu.sync_copy(acc.at[pl.ds(s * RZ, RZ)],
                            out_h.at[c, h, pl.ds(s * RZ, RZ)])
            plsc.subcore_barrier()

    return kern(srcp, dstp, ex, den0, den1, *hs_list, zeros)


def _combine_parts(parts, gb, nd):
    BN = 1000

    def body(p_r, gb_r, o_r):
        o_r[...] = jnp.concatenate([p_r[0, h] + p_r[1, h]
                                    for h in range(HEADS)], axis=-1) + gb_r[...]

    return pl.pallas_call(
        body,
        grid=(nd // BN,),
        in_specs=[
            pl.BlockSpec((NC, HEADS, BN, HID), lambda i: (0, 0, i, 0)),
            pl.BlockSpec((1, F), lambda i: (0, 0)),
        ],
        out_specs=pl.BlockSpec((BN, F), lambda i: (i, 0)),
        out_shape=jax.ShapeDtypeStruct((nd, F), jnp.float32),
    )(parts, gb[None])


def _gat_mlp(o, mp, res):
    nd = res.shape[0]

    def body(o_ref, W1, b1, g1, be1, W2, b2, g2, be2, res_r, o_r):
        o = o_ref[...]
        h = jnp.dot(o, W1[...], preferred_element_type=jnp.float32) + b1[...]
        m = jnp.mean(h, axis=0, keepdims=True)
        v = jnp.mean((h - m) ** 2, axis=0, keepdims=True)
        h = jnp.maximum(g1[...] * (h - m) / jnp.sqrt(v + 1e-5) + be1[...], 0.0)
        h = jnp.dot(h, W2[...], preferred_element_type=jnp.float32) + b2[...]
        m = jnp.mean(h, axis=0, keepdims=True)
        v = jnp.mean((h - m) ** 2, axis=0, keepdims=True)
        h = jnp.maximum(g2[...] * (h - m) / jnp.sqrt(v + 1e-5) + be2[...], 0.0)
        o_r[...] = h + res_r[...]

    return pl.pallas_call(
        body, out_shape=jax.ShapeDtypeStruct((nd, HID), jnp.float32),
    )(o,
      mp['W1'], mp['b1'][None], mp['g1'][None], mp['be1'][None],
      mp['W2'], mp['b2'][None], mp['g2'][None], mp['be2'][None], res)


def _gat_block(xs, xd, ei, gp, mlp_p, res):
    ns = xs.shape[0]
    E = ei.shape[1]
    nd = xd.shape[0]
    Ep = ((E + NW * 64 - 1) // (NW * 64)) * (NW * 64)
    if Ep == E:
        Ep += NW * 64
    pad = Ep - E
    srcp = jnp.concatenate([ei[0], jnp.full((pad,), ns, jnp.int32)])
    dstp = jnp.concatenate([ei[1], jnp.zeros((pad,), jnp.int32)])
    hs0, hs1, hs2, hs3, asrc16, adst16, g16 = _gat_prep(
        xs, xd, gp['Ws'], gp['Wd'], gp['as'], gp['ad'])
    ex, den = _gat_att(srcp, dstp, asrc16, adst16, jnp.reshape(g16, (16,)), nd)
    parts = _gat_scatter(srcp, dstp, ex, den[0], den[1],
                         (hs0, hs1, hs2, hs3), nd)
    o = _combine_parts(parts, gp['b'], nd)
    return _gat_mlp(o, mlp_p, res)


def kernel(x, edge_index, edge_attr, x_cl, c2c_edge_index, c2c_edge_attr,
           atom2c_edge_index, c2atom_edge_index, params):
    src = edge_index[0]
    dst = edge_index[1]
    for p in params['atom_convs']:
        parts = _gine_edges(x, src, dst, edge_attr)
        x = _conv_mlp(x, parts, p)
    h = _gat_block(x_cl, x, c2atom_edge_index, params['unpool'],
                   params['c2atom_mlp'], x)
    h_cl = _gat_block(x, x_cl, atom2c_edge_index, params['pool'],
                      params['atom2c_mlp'], x_cl)
    return (h, h_cl)


# speedup vs baseline: 4.4511x; 1.5302x over previous
import functools

import jax
import jax.numpy as jnp
from jax import lax
from jax.experimental import pallas as pl
from jax.experimental.pallas import tpu as pltpu
from jax.experimental.pallas import tpu_sc as plsc

NC, NS, L = 2, 16, 16
NW = NC * NS
HID = 128
HEADS = 4
F = HEADS * HID


def _gine_edges(x, src, dst, ea):
    N = x.shape[0]
    E = src.shape[0]
    EPT = E // NW
    K = 80
    CH = EPT // K
    NP = -(-N // (NS * 8)) * (NS * 8)
    ROWS = NP // NS
    mesh = plsc.VectorSubcoreMesh(core_axis_name="c", subcore_axis_name="s",
                                  num_cores=NC, num_subcores=NS)

    @functools.partial(
        pl.kernel, mesh=mesh,
        out_type=jax.ShapeDtypeStruct((NC, NP, HID), jnp.float32),
        scratch_types=[
            pltpu.VMEM((K,), jnp.int32), pltpu.VMEM((K,), jnp.int32),
            pltpu.VMEM((K,), jnp.int32), pltpu.VMEM((K,), jnp.int32),
            pltpu.VMEM((K, HID), jnp.float32),
            pltpu.VMEM((K, HID), jnp.float32),
            pltpu.VMEM((K, HID), jnp.float32),
            pltpu.VMEM((K, HID), jnp.float32),
            pltpu.VMEM_SHARED((NP, HID), jnp.float32),
            pltpu.SemaphoreType.DMA, pltpu.SemaphoreType.DMA,
        ])
    def kern(x_h, src_h, dst_h, ea_h, out_h,
             sv0, sv1, dv0, dv1, xg0, xg1, ea0, ea1, acc, g0, g1):
        c = lax.axis_index("c")
        s = lax.axis_index("s")
        tid = c * NS + s
        srcv = (sv0, sv1)
        dstv = (dv0, dv1)
        xgv = (xg0, xg1)
        eav = (ea0, ea1)
        gsem = (g0, g1)

        def zrow(r, _):
            for cc in range(HID // L):
                xg0[r, pl.ds(cc * L, L)] = jnp.zeros((L,), jnp.float32)
            return 0

        lax.fori_loop(0, K, zrow, 0)
        nzc = ROWS // K
        for j in range(nzc):
            pltpu.sync_copy(xg0, acc.at[pl.ds(s * ROWS + j * K, K)])
        rem = ROWS - nzc * K
        if rem:
            pltpu.sync_copy(xg0.at[pl.ds(0, rem)],
                            acc.at[pl.ds(s * ROWS + nzc * K, rem)])
        plsc.subcore_barrier()
        base0 = tid * EPT

        def prefetch(slot, ci):
            b = base0 + ci * K
            pltpu.sync_copy(src_h.at[pl.ds(b, K)], srcv[slot])
            pltpu.sync_copy(dst_h.at[pl.ds(b, K)], dstv[slot])
            pltpu.async_copy(x_h.at[srcv[slot]], xgv[slot], gsem[slot])
            pltpu.async_copy(ea_h.at[pl.ds(b, K)], eav[slot], gsem[slot])

        def process(slot):
            pltpu.make_async_copy(x_h.at[srcv[slot]], xgv[slot], gsem[slot]).wait()
            pltpu.make_async_copy(ea_h.at[pl.ds(0, K)], eav[slot], gsem[slot]).wait()
            xgs = xgv[slot]
            eas = eav[slot]

            def grp(g_, _):
                for rr in range(4):
                    r = g_ * 4 + rr
                    for cc in range(HID // L):
                        sl = pl.ds(cc * L, L)
                        xgs[r, sl] = jnp.maximum(xgs[r, sl] + eas[r, sl], 0.0)
                return 0

            lax.fori_loop(0, K // 4, grp, 0)
            pltpu.sync_copy(xgs, acc.at[dstv[slot]], add=True)

        prefetch(0, 0)

        def t_body(t, _):
            for j in range(2):
                ci = 2 * t + j
                prefetch(1 - j, jnp.minimum(ci + 1, CH - 1))
                process(j)
            return 0

        lax.fori_loop(0, CH // 2, t_body, 0)
        process(0)
        plsc.subcore_barrier()
        pltpu.sync_copy(acc.at[pl.ds(s * ROWS, ROWS)],
                        out_h.at[c, pl.ds(s * ROWS, ROWS)])

    return kern(x, src, dst, ea)


def _conv_mlp(x, parts, mp):
    N = x.shape[0]

    def body(x_r, p_r, e_r, W1, b1, g1, be1, W2, b2, g2, be2, o_r):
        z = (1.0 + e_r[0, 0]) * x_r[...] + p_r[0, :N] + p_r[1, :N]
        h = jnp.dot(z, W1[...], preferred_element_type=jnp.float32) + b1[...]
        m = jnp.mean(h, axis=0, keepdims=True)
        v = jnp.mean((h - m) ** 2, axis=0, keepdims=True)
        h = jnp.maximum(g1[...] * (h - m) / jnp.sqrt(v + 1e-5) + be1[...], 0.0)
        h = jnp.dot(h, W2[...], preferred_element_type=jnp.float32) + b2[...]
        m = jnp.mean(h, axis=0, keepdims=True)
        v = jnp.mean((h - m) ** 2, axis=0, keepdims=True)
        h = jnp.maximum(g2[...] * (h - m) / jnp.sqrt(v + 1e-5) + be2[...], 0.0)
        o_r[...] = h + x_r[...]

    p = mp['mlp']
    return pl.pallas_call(
        body, out_shape=jax.ShapeDtypeStruct((N, HID), jnp.float32),
    )(x, parts, jnp.reshape(mp['eps'], (1, 1)),
      p['W1'], p['b1'][None], p['g1'][None], p['be1'][None],
      p['W2'], p['b2'][None], p['g2'][None], p['be2'][None])


def _gat_prep(xs, xd, Ws, Wd, a_s, a_d):
    ns, nd = xs.shape[0], xd.shape[0]

    def body(xs_r, xd_r, Ws_r, Wd_r, as_r, ad_r, hs0, hs1, hs2, hs3,
             asr_o, adr_o, g_o):
        xsv = xs_r[...]
        xdv = xd_r[...]
        Wsv = Ws_r[...]
        Wdv = Wd_r[...]
        zrow = jnp.zeros((8, HID), jnp.float32)
        for h, hs_o in enumerate((hs0, hs1, hs2, hs3)):
            blk = jnp.dot(xsv, Wsv[:, h * HID:(h + 1) * HID],
                          preferred_element_type=jnp.float32)
            hs_o[...] = jnp.concatenate([blk, zrow], axis=0)
        zc = jnp.zeros((HID, 16 - HEADS), jnp.float32)
        wa = [jnp.dot(Wsv[:, h * HID:(h + 1) * HID],
                      jnp.reshape(as_r[h, :], (HID, 1)),
                      preferred_element_type=jnp.float32) for h in range(HEADS)]
        wd = [jnp.dot(Wdv[:, h * HID:(h + 1) * HID],
                      jnp.reshape(ad_r[h, :], (HID, 1)),
                      preferred_element_type=jnp.float32) for h in range(HEADS)]
        Wa = jnp.concatenate(wa + [zc], axis=1)
        Wd16 = jnp.concatenate(wd + [zc], axis=1)
        asrc = jnp.dot(xsv, Wa, preferred_element_type=jnp.float32)
        adst = jnp.dot(xdv, Wd16, preferred_element_type=jnp.float32)
        asr_o[...] = jnp.concatenate(
            [jnp.concatenate([asrc, jnp.zeros((ns, HID - 16), jnp.float32)],
                             axis=1),
             jnp.full((8, HID), -1e30, jnp.float32)], axis=0)
        adr_o[...] = jnp.concatenate(
            [adst, jnp.zeros((nd, HID - 16), jnp.float32)], axis=1)
        y = jnp.max(asrc, axis=0, keepdims=True) + jnp.max(adst, axis=0,
                                                           keepdims=True)
        g_o[...] = jnp.where(y > 0, y, 0.2 * y)

    return pl.pallas_call(
        body,
        out_shape=(
            jax.ShapeDtypeStruct((ns + 8, HID), jnp.float32),
            jax.ShapeDtypeStruct((ns + 8, HID), jnp.float32),
            jax.ShapeDtypeStruct((ns + 8, HID), jnp.float32),
            jax.ShapeDtypeStruct((ns + 8, HID), jnp.float32),
            jax.ShapeDtypeStruct((ns + 8, HID), jnp.float32),
            jax.ShapeDtypeStruct((nd, HID), jnp.float32),
            jax.ShapeDtypeStruct((1, 16), jnp.float32),
        ),
    )(xs, xd, Ws, Wd, a_s, a_d)


def _gat_att(srcp, dstp, asrc16, adst16, g16, nd):
    Ep = srcp.shape[0]
    EPT = Ep // NW
    K = 64
    CH = EPT // K
    ndp = -(-nd // (NS * 8)) * (NS * 8)
    RZ = ndp // NS
    zeros = jnp.zeros((ndp, HID), jnp.float32)
    mesh = plsc.VectorSubcoreMesh(core_axis_name="c", subcore_axis_name="s",
                                  num_cores=NC, num_subcores=NS)

    @functools.partial(
        pl.kernel, mesh=mesh,
        out_type=(jax.ShapeDtypeStruct((Ep, HID), jnp.float32),
                  jax.ShapeDtypeStruct((NC, ndp, HID), jnp.float32)),
        scratch_types=[
            pltpu.VMEM((K,), jnp.int32),
            pltpu.VMEM((K,), jnp.int32),
            pltpu.VMEM((K, HID), jnp.float32),
            pltpu.VMEM((K, HID), jnp.float32),
            pltpu.VMEM((K, HID), jnp.float32),
            pltpu.VMEM((16,), jnp.float32),
            pltpu.VMEM_SHARED((ndp, HID), jnp.float32),
            pltpu.SemaphoreType.DMA,
        ])
    def kern(src_h, dst_h, asrc_h, adst_h, g_h, z_h, ex_h, den_h,
             src_v, dst_v, sg, dg, exb, gv, den, sem):
        c = lax.axis_index("c")
        s = lax.axis_index("s")
        tid = c * NS + s
        pltpu.sync_copy(g_h, gv)
        pltpu.sync_copy(z_h.at[pl.ds(s * RZ, RZ)], den.at[pl.ds(s * RZ, RZ)])
        pltpu.sync_copy(z_h.at[pl.ds(0, K)], exb)
        plsc.subcore_barrier()
        gvec = gv[...]
        base0 = tid * EPT

        def chunk(i, _):
            b = base0 + i * K
            pltpu.sync_copy(src_h.at[pl.ds(b, K)], src_v)
            pltpu.sync_copy(dst_h.at[pl.ds(b, K)], dst_v)
            pltpu.async_copy(asrc_h.at[src_v], sg, sem).wait()
            pltpu.async_copy(adst_h.at[dst_v], dg, sem).wait()

            def row(r, _):
                y = sg[r, pl.ds(0, 16)] + dg[r, pl.ds(0, 16)]
                al = jnp.where(y > 0, y, 0.2 * y)
                exb[r, pl.ds(0, 16)] = jnp.exp(al - gvec)
                return 0

            lax.fori_loop(0, K, row, 0)
            pltpu.sync_copy(exb, ex_h.at[pl.ds(b, K)])
            pltpu.sync_copy(exb, den.at[dst_v], add=True)
            return 0

        lax.fori_loop(0, CH, chunk, 0)
        plsc.subcore_barrier()
        pltpu.sync_copy(den.at[pl.ds(s * RZ, RZ)], den_h.at[c, pl.ds(s * RZ, RZ)])

    return kern(srcp, dstp, asrc16, adst16, g16, zeros)


def _gat_scatter(srcp, dstp, ex, den0, den1, hs_list, nd):
    Ep = srcp.shape[0]
    EPT = Ep // NW
    K = 64
    CH = EPT // K
    ndp = -(-nd // (NS * 8)) * (NS * 8)
    RZ = ndp // NS
    zeros = jnp.zeros((ndp, HID), jnp.float32)
    mesh = plsc.VectorSubcoreMesh(core_axis_name="c", subcore_axis_name="s",
                                  num_cores=NC, num_subcores=NS)

    @functools.partial(
        pl.kernel, mesh=mesh,
        out_type=jax.ShapeDtypeStruct((NC, HEADS, ndp, HID), jnp.float32),
        scratch_types=[
            pltpu.VMEM((K,), jnp.int32),
            pltpu.VMEM((K,), jnp.int32),
            pltpu.VMEM((K, HID), jnp.float32),
            pltpu.VMEM((K, HID), jnp.float32),
            pltpu.VMEM((K, HID), jnp.float32),
            pltpu.VMEM((K, HID), jnp.float32),
            pltpu.VMEM((K, HID), jnp.float32),
            pltpu.VMEM((K, 16), jnp.float32),
            pltpu.VMEM_SHARED((ndp, HID), jnp.float32),
            pltpu.SemaphoreType.DMA,
        ])
    def kern(src_h, dst_h, ex_h, d0_h, d1_h, hs0, hs1, hs2, hs3, z_h, out_h,
             src_v, dst_v, exv, d0, d1, hg, ob, wb, acc, sem):
        c = lax.axis_index("c")
        s = lax.axis_index("s")
        tid = c * NS + s
        base0 = tid * EPT
        hs_h = (hs0, hs1, hs2, hs3)
        for h in range(HEADS):
            pltpu.sync_copy(z_h.at[pl.ds(s * RZ, RZ)], acc.at[pl.ds(s * RZ, RZ)])
            plsc.subcore_barrier()

            def chunk(i, _, h=h):
                b = base0 + i * K
                pltpu.sync_copy(src_h.at[pl.ds(b, K)], src_v)
                pltpu.sync_copy(dst_h.at[pl.ds(b, K)], dst_v)
                pltpu.sync_copy(ex_h.at[pl.ds(b, K)], exv)
                pltpu.async_copy(d0_h.at[dst_v], d0, sem).wait()
                pltpu.async_copy(d1_h.at[dst_v], d1, sem).wait()
                pltpu.async_copy(hs_h[h].at[src_v], hg, sem).wait()

                def row(r, _, h=h):
                    den = jnp.maximum(d0[r, pl.ds(0, 16)] + d1[r, pl.ds(0, 16)],
                                      1e-30)
                    w = exv[r, pl.ds(0, 16)] / den
                    ws = w[h]
                    for cc in range(HID // L):
                        sl = pl.ds(cc * L, L)
                        ob[r, sl] = hg[r, sl] * ws
                    return 0

                lax.fori_loop(0, K, row, 0)
                pltpu.sync_copy(ob, acc.at[dst_v], add=True)
                return 0

            lax.fori_loop(0, CH, chunk, 0)
            plsc.subcore_barrier()
            pltpu.sync_copy(acc.at[pl.ds(s * RZ, RZ)],
                            out_h.at[c, h, pl.ds(s * RZ, RZ)])
            plsc.subcore_barrier()

    return kern(srcp, dstp, ex, den0, den1, *hs_list, zeros)


def _combine_parts(parts, gb, nd):
    BN = 1000

    def body(p_r, gb_r, o_r):
        o_r[...] = jnp.concatenate([p_r[0, h] + p_r[1, h]
                                    for h in range(HEADS)], axis=-1) + gb_r[...]

    return pl.pallas_call(
        body,
        grid=(nd // BN,),
        in_specs=[
            pl.BlockSpec((NC, HEADS, BN, HID), lambda i: (0, 0, i, 0)),
            pl.BlockSpec((1, F), lambda i: (0, 0)),
        ],
        out_specs=pl.BlockSpec((BN, F), lambda i: (i, 0)),
        out_shape=jax.ShapeDtypeStruct((nd, F), jnp.float32),
    )(parts, gb[None])


def _gat_mlp(o, mp, res):
    nd = res.shape[0]

    def body(o_ref, W1, b1, g1, be1, W2, b2, g2, be2, res_r, o_r):
        o = o_ref[...]
        h = jnp.dot(o, W1[...], preferred_element_type=jnp.float32) + b1[...]
        m = jnp.mean(h, axis=0, keepdims=True)
        v = jnp.mean((h - m) ** 2, axis=0, keepdims=True)
        h = jnp.maximum(g1[...] * (h - m) / jnp.sqrt(v + 1e-5) + be1[...], 0.0)
        h = jnp.dot(h, W2[...], preferred_element_type=jnp.float32) + b2[...]
        m = jnp.mean(h, axis=0, keepdims=True)
        v = jnp.mean((h - m) ** 2, axis=0, keepdims=True)
        h = jnp.maximum(g2[...] * (h - m) / jnp.sqrt(v + 1e-5) + be2[...], 0.0)
        o_r[...] = h + res_r[...]

    return pl.pallas_call(
        body, out_shape=jax.ShapeDtypeStruct((nd, HID), jnp.float32),
    )(o,
      mp['W1'], mp['b1'][None], mp['g1'][None], mp['be1'][None],
      mp['W2'], mp['b2'][None], mp['g2'][None], mp['be2'][None], res)


def _gat_block(xs, xd, ei, gp, mlp_p, res):
    ns = xs.shape[0]
    E = ei.shape[1]
    nd = xd.shape[0]
    Ep = ((E + NW * 64 - 1) // (NW * 64)) * (NW * 64)
    if Ep == E:
        Ep += NW * 64
    pad = Ep - E
    srcp = jnp.concatenate([ei[0], jnp.full((pad,), ns, jnp.int32)])
    dstp = jnp.concatenate([ei[1], jnp.zeros((pad,), jnp.int32)])
    hs0, hs1, hs2, hs3, asrc16, adst16, g16 = _gat_prep(
        xs, xd, gp['Ws'], gp['Wd'], gp['as'], gp['ad'])
    ex, den = _gat_att(srcp, dstp, asrc16, adst16, jnp.reshape(g16, (16,)), nd)
    parts = _gat_scatter(srcp, dstp, ex, den[0], den[1],
                         (hs0, hs1, hs2, hs3), nd)
    o = _combine_parts(parts, gp['b'], nd)
    return _gat_mlp(o, mlp_p, res)


def kernel(x, edge_index, edge_attr, x_cl, c2c_edge_index, c2c_edge_attr,
           atom2c_edge_index, c2atom_edge_index, params):
    for p in params['atom_convs']:
        parts = _gine_edges(x, edge_index[0], edge_index[1], edge_attr)
        x = _conv_mlp(x, parts, p)
    h = _gat_block(x_cl, x, c2atom_edge_index, params['unpool'],
                   params['c2atom_mlp'], x)
    h_cl = _gat_block(x, x_cl, atom2c_edge_index, params['pool'],
                      params['atom2c_mlp'], x_cl)
    return (h, h_cl)
